# baseline (speedup 1.0000x reference)
import functools

import jax
import jax.numpy as jnp
from jax import lax
from jax.experimental import pallas as pl
from jax.experimental.pallas import tpu as pltpu
from jax.experimental.pallas import tpu_sc as plsc

_K = 2


def _make_sc_gate(B, C, HW, E):
    BC = B * C
    NW = 16
    rows_per = BC // NW
    mesh = plsc.VectorSubcoreMesh(core_axis_name="c", subcore_axis_name="s")
    L = 16

    @functools.partial(
        pl.kernel,
        mesh=mesh,
        compiler_params=pltpu.CompilerParams(needs_layout_passes=False),
        out_type=[
            jax.ShapeDtypeStruct((L,), jnp.int32),
            jax.ShapeDtypeStruct((L,), jnp.float32),
            jax.ShapeDtypeStruct((L,), jnp.float32),
        ],
        scratch_types=[
            pltpu.VMEM((rows_per, HW), jnp.float32),
            pltpu.VMEM((((rows_per + L - 1) // L) * L,), jnp.float32),
            pltpu.VMEM_SHARED((BC,), jnp.float32),
            pltpu.VMEM((BC,), jnp.float32),
            pltpu.VMEM((BC,), jnp.float32),
            pltpu.VMEM((L,), jnp.int32),
            pltpu.VMEM((L,), jnp.float32),
            pltpu.VMEM((L,), jnp.float32),
        ],
    )
    def sc_gate(x_hbm, wg_hbm, idx_hbm, gv_hbm, loss_hbm,
                xbuf, mbuf, gx_sh, gxv, wgv, idx_v, gv_v, loss_v):
        core = lax.axis_index("c")
        sid = lax.axis_index("s")
        base = sid * rows_per
        lane0 = lax.iota(jnp.int32, L)

        @pl.when(core == 0)
        def _():
            pltpu.sync_copy(x_hbm.at[pl.ds(base, rows_per)], xbuf)
            mvs = [jnp.zeros((L,), jnp.float32)
                   for _ in range((rows_per + L - 1) // L)]
            for r in range(rows_per):
                acc = jnp.zeros((L,), jnp.float32)
                for i in range(HW // L):
                    acc = acc + xbuf[r, pl.ds(i * L, L)]
                m = jnp.sum(acc) * (1.0 / HW)
                mvs[r // L] = mvs[r // L] + jnp.where(lane0 == r % L, m, 0.0)
            for q, mv in enumerate(mvs):
                mbuf[pl.ds(q * L, L)] = mv
            pltpu.sync_copy(mbuf.at[pl.ds(0, rows_per)],
                            gx_sh.at[pl.ds(base, rows_per)])
            plsc.subcore_barrier()

        @pl.when((core == 0) & (sid == 0))
        def _():
            pltpu.sync_copy(gx_sh, gxv)
            pltpu.sync_copy(wg_hbm, wgv)
            lane = lax.iota(jnp.int32, L)
            lo = lane < E
            fold_idx = jnp.where(lane + E < L, lane + E, 0)

            imp0 = jnp.zeros((L,), jnp.float32)
            load0 = jnp.zeros((L,), jnp.float32)
            iv0 = jnp.zeros((L,), jnp.int32)
            gv0 = jnp.zeros((L,), jnp.float32)

            def sample(b, carry):
                imp, loadv, iv, gv = carry
                accw = jnp.zeros((L,), jnp.float32)

                def chan16(c16, a):
                    gch = gxv[pl.ds(b * C + c16 * L, L)]
                    for q in range(0, L, 2):
                        sv = jnp.where(lo, gch[q], gch[q + 1])
                        a = a + sv * wgv[pl.ds((c16 * L + q) * E, L)]
                    return a

                accw = lax.fori_loop(0, C // L, chan16, accw)
                folded = accw + jnp.take(accw, fold_idx)
                logits = jnp.where(lo, folded, -jnp.inf)
                m = jnp.max(logits)
                ex = jnp.exp(logits - m)
                s = jnp.sum(jnp.where(lo, ex, 0.0))
                sm = ex / jnp.full((L,), s)

                v1 = jnp.max(sm)
                i1 = jnp.min(jnp.where(sm == v1, lane, 99))
                sm2 = jnp.where(lane == i1, -jnp.inf, sm)
                v2 = jnp.max(sm2)
                i2 = jnp.min(jnp.where(sm2 == v2, lane, 99))
                denv = jnp.full((L,), v1 + v2 + 1e-6)
                g1v = jnp.full((L,), v1) / denv
                g2v = jnp.full((L,), v2) / denv
                dense = (jnp.where(lane == i1, g1v, 0.0)
                         + jnp.where(lane == i2, g2v, 0.0))
                imp = imp + dense
                loadv = loadv + jnp.where(dense > 0.0, 1.0, 0.0)
                iv = (iv + jnp.where(lane == 2 * b, i1, 0)
                      + jnp.where(lane == 2 * b + 1, i2, 0))
                gv = (gv + jnp.where(lane == 2 * b, g1v, 0.0)
                      + jnp.where(lane == 2 * b + 1, g2v, 0.0))
                return imp, loadv, iv, gv

            imp, loadv, iv, gv = lax.fori_loop(
                0, B, sample, (imp0, load0, iv0, gv0))

            def cv_sq(v):
                mu = jnp.sum(v) * (1.0 / E)
                d = jnp.where(lo, v - mu, 0.0)
                varv = jnp.full((L,), jnp.sum(d * d) * (1.0 / (E - 1)))
                return varv / jnp.full((L,), mu * mu + 1e-10)

            lossv = (cv_sq(imp) + cv_sq(loadv)) * 1e-2
            idx_v[...] = iv
            gv_v[...] = gv
            loss_v[...] = jnp.where(lane == 0, lossv, 0.0)
            pltpu.sync_copy(idx_v, idx_hbm)
            pltpu.sync_copy(gv_v, gv_hbm)
            pltpu.sync_copy(loss_v, loss_hbm)

    return sc_gate


def _conv_body(x_ref, idx_ref, gvv_ref, wf_ref, beta_ref, o_ref,
               xt_s, xts_s, w_ref):
    B = x_ref.shape[0]
    f32 = jnp.float32
    bf16 = jnp.bfloat16

    lane16 = jax.lax.broadcasted_iota(jnp.int32, (1, 16), 1)
    iv = idx_ref[...]
    gvv = gvv_ref[...]

    bv = beta_ref[...]
    ri = jax.lax.broadcasted_iota(jnp.int32, bv.shape, 0)
    ci_ = jax.lax.broadcasted_iota(jnp.int32, bv.shape, 1)
    W = 16
    HW = x_ref.shape[2]
    lane320 = jax.lax.broadcasted_iota(jnp.int32, (1, HW + 4 * W), 1) % W

    xtall = jnp.tanh(x_ref[...])
    xt_s[...] = xtall
    xts_s[...] = (xtall * jax.nn.sigmoid(xtall)).astype(bf16)
    w_ref[...] = wf_ref[...].astype(bf16)

    C = x_ref.shape[1]
    cb0 = jnp.full((C, HW), 0.7310586, bf16)

    def sample(b, carry):
        xt = xt_s[b]
        xt2 = xt * xt
        xts = xts_s[b]

        def poly23(e_):
            b2 = 2.25 * jnp.sum(jnp.where((ri == e_) & (ci_ == 1), bv, 0.0))
            b3 = (300.0 / 9.0) * jnp.sum(
                jnp.where((ri == e_) & (ci_ == 2), bv, 0.0))
            p2 = xt2 - b2
            p3 = xt * p2 - b3 * xt
            return p2, p3

        e0 = jnp.sum(jnp.where(lane16 == 2 * b, iv, 0))
        e1 = jnp.sum(jnp.where(lane16 == 2 * b + 1, iv, 0))
        gate0 = jnp.sum(jnp.where(lane16 == 2 * b, gvv, 0.0))
        gate1 = jnp.sum(jnp.where(lane16 == 2 * b + 1, gvv, 0.0))
        p2a, p3a = poly23(e0)
        p2b, p3b = poly23(e1)
        pp = jnp.concatenate([p2a, p3a, p2b, p3b], axis=0)
        pp = (pp * jax.nn.sigmoid(pp)).astype(bf16)

        acc = jnp.zeros((o_ref.shape[1], HW), f32)
        for k in range(_K):
            e = e0 if k == 0 else e1
            gate = gate0 if k == 0 else gate1
            s23 = jax.lax.slice(pp, (2 * C * k, 0), (2 * C * (k + 1), HW))
            g = jnp.concatenate([cb0, xts, s23], axis=0)

            CI = g.shape[0]
            padz = jnp.zeros((CI, 2 * W), dtype=bf16)
            gext = jnp.concatenate([padz, g, padz], axis=1)
            gl = jnp.where(lane320 != W - 1, gext, bf16(0))
            gr = jnp.where(lane320 != 0, gext, bf16(0))

            acck = jnp.zeros((o_ref.shape[1], HW), f32)
            acck2 = jnp.zeros((o_ref.shape[1], HW), f32)
            for j in range(9):
                dy, dx = j // 3, j % 3
                off = W * (dy - 1) + (dx - 1)
                src = (gl, gext, gr)[dx]
                s = jax.lax.slice(src, (0, 2 * W + off),
                                  (CI, 2 * W + off + HW))
                d = jax.lax.dot(w_ref[e, j], s, preferred_element_type=f32)
                if j % 2 == 0:
                    acck = acck + d
                else:
                    acck2 = acck2 + d
            acc = acc + gate * (acck + acck2)

        o_ref[b] = acc
        return carry

    jax.lax.fori_loop(0, B, sample, 0)


def kernel(x, w_gate, poly_weights, beta_weights):
    B, C, H, W = x.shape
    E, O, CI, KH, KW = poly_weights.shape
    HW = H * W
    x2 = x.reshape(B, C, HW)
    pwt = jnp.transpose(poly_weights, (0, 3, 4, 1, 2)).reshape(
        E, KH * KW, O, CI)

    sc_gate = _make_sc_gate(B, C, HW, E)
    idx16, gv16, loss16 = sc_gate(x2.reshape(B * C, HW), w_gate.reshape(-1))

    y = pl.pallas_call(
        _conv_body,
        out_shape=jax.ShapeDtypeStruct((B, O, HW), jnp.float32),
        scratch_shapes=[
            pltpu.VMEM((B, C, HW), jnp.float32),
            pltpu.VMEM((B, C, HW), jnp.bfloat16),
            pltpu.VMEM((E, KH * KW, O, CI), jnp.bfloat16),
        ],
    )(x2, idx16.reshape(1, 16), gv16.reshape(1, 16), pwt, beta_weights)

    return y.reshape(B, O, H, W), loss16[0]


# speedup vs baseline: 1.0017x; 1.0017x over previous
import functools

import jax
import jax.numpy as jnp
from jax import lax
from jax.experimental import pallas as pl
from jax.experimental.pallas import tpu as pltpu
from jax.experimental.pallas import tpu_sc as plsc

_K = 2


def _make_sc_gate(B, C, HW, E):
    BC = B * C
    NW = 16
    rows_per = BC // NW
    mesh = plsc.VectorSubcoreMesh(core_axis_name="c", subcore_axis_name="s")
    L = 16

    @functools.partial(
        pl.kernel,
        mesh=mesh,
        compiler_params=pltpu.CompilerParams(needs_layout_passes=False),
        out_type=[
            jax.ShapeDtypeStruct((L,), jnp.int32),
            jax.ShapeDtypeStruct((L,), jnp.float32),
            jax.ShapeDtypeStruct((L,), jnp.float32),
        ],
        scratch_types=[
            pltpu.VMEM((rows_per, HW), jnp.float32),
            pltpu.VMEM((((rows_per + L - 1) // L) * L,), jnp.float32),
            pltpu.VMEM_SHARED((BC,), jnp.float32),
            pltpu.VMEM((BC,), jnp.float32),
            pltpu.VMEM((BC,), jnp.float32),
            pltpu.VMEM((L,), jnp.int32),
            pltpu.VMEM((L,), jnp.float32),
            pltpu.VMEM((L,), jnp.float32),
        ],
    )
    def sc_gate(x_hbm, wg_hbm, idx_hbm, gv_hbm, loss_hbm,
                xbuf, mbuf, gx_sh, gxv, wgv, idx_v, gv_v, loss_v):
        core = lax.axis_index("c")
        sid = lax.axis_index("s")
        base = sid * rows_per
        lane0 = lax.iota(jnp.int32, L)

        @pl.when(core == 0)
        def _():
            pltpu.sync_copy(x_hbm.at[pl.ds(base, rows_per)], xbuf)
            mvs = [jnp.zeros((L,), jnp.float32)
                   for _ in range((rows_per + L - 1) // L)]
            for r in range(rows_per):
                acc = jnp.zeros((L,), jnp.float32)
                for i in range(HW // L):
                    acc = acc + xbuf[r, pl.ds(i * L, L)]
                m = jnp.sum(acc) * (1.0 / HW)
                mvs[r // L] = mvs[r // L] + jnp.where(lane0 == r % L, m, 0.0)
            for q, mv in enumerate(mvs):
                mbuf[pl.ds(q * L, L)] = mv
            pltpu.sync_copy(mbuf.at[pl.ds(0, rows_per)],
                            gx_sh.at[pl.ds(base, rows_per)])
            plsc.subcore_barrier()

        @pl.when((core == 0) & (sid == 0))
        def _():
            pltpu.sync_copy(gx_sh, gxv)
            pltpu.sync_copy(wg_hbm, wgv)
            lane = lax.iota(jnp.int32, L)
            lo = lane < E
            fold_idx = jnp.where(lane + E < L, lane + E, 0)

            imp0 = jnp.zeros((L,), jnp.float32)
            load0 = jnp.zeros((L,), jnp.float32)
            iv0 = jnp.zeros((L,), jnp.int32)
            gv0 = jnp.zeros((L,), jnp.float32)

            def sample(b, carry):
                imp, loadv, iv, gv = carry
                accw = jnp.zeros((L,), jnp.float32)

                def chan16(c16, a):
                    gch = gxv[pl.ds(b * C + c16 * L, L)]
                    for q in range(0, L, 2):
                        sv = jnp.where(lo, gch[q], gch[q + 1])
                        a = a + sv * wgv[pl.ds((c16 * L + q) * E, L)]
                    return a

                accw = lax.fori_loop(0, C // L, chan16, accw)
                folded = accw + jnp.take(accw, fold_idx)
                logits = jnp.where(lo, folded, -jnp.inf)
                m = jnp.max(logits)
                ex = jnp.exp(logits - m)
                s = jnp.sum(jnp.where(lo, ex, 0.0))
                sm = ex / jnp.full((L,), s)

                v1 = jnp.max(sm)
                i1 = jnp.min(jnp.where(sm == v1, lane, 99))
                sm2 = jnp.where(lane == i1, -jnp.inf, sm)
                v2 = jnp.max(sm2)
                i2 = jnp.min(jnp.where(sm2 == v2, lane, 99))
                denv = jnp.full((L,), v1 + v2 + 1e-6)
                g1v = jnp.full((L,), v1) / denv
                g2v = jnp.full((L,), v2) / denv
                dense = (jnp.where(lane == i1, g1v, 0.0)
                         + jnp.where(lane == i2, g2v, 0.0))
                imp = imp + dense
                loadv = loadv + jnp.where(dense > 0.0, 1.0, 0.0)
                iv = (iv + jnp.where(lane == 2 * b, i1, 0)
                      + jnp.where(lane == 2 * b + 1, i2, 0))
                gv = (gv + jnp.where(lane == 2 * b, g1v, 0.0)
                      + jnp.where(lane == 2 * b + 1, g2v, 0.0))
                return imp, loadv, iv, gv

            imp, loadv, iv, gv = lax.fori_loop(
                0, B, sample, (imp0, load0, iv0, gv0))

            def cv_sq(v):
                mu = jnp.sum(v) * (1.0 / E)
                d = jnp.where(lo, v - mu, 0.0)
                varv = jnp.full((L,), jnp.sum(d * d) * (1.0 / (E - 1)))
                return varv / jnp.full((L,), mu * mu + 1e-10)

            lossv = (cv_sq(imp) + cv_sq(loadv)) * 1e-2
            idx_v[...] = iv
            gv_v[...] = gv
            loss_v[...] = jnp.where(lane == 0, lossv, 0.0)
            pltpu.sync_copy(idx_v, idx_hbm)
            pltpu.sync_copy(gv_v, gv_hbm)
            pltpu.sync_copy(loss_v, loss_hbm)

    return sc_gate


def _conv_body(x_ref, idx_ref, gvv_ref, wf_ref, beta_ref, o_ref,
               xt_s, xts_s, w_ref):
    B = o_ref.shape[0]
    f32 = jnp.float32
    bf16 = jnp.bfloat16

    lane16 = jax.lax.broadcasted_iota(jnp.int32, (1, 16), 1)
    iv = idx_ref[...]
    gvv = gvv_ref[...]

    bv = beta_ref[...]
    ri = jax.lax.broadcasted_iota(jnp.int32, bv.shape, 0)
    ci_ = jax.lax.broadcasted_iota(jnp.int32, bv.shape, 1)
    W = 16
    HW = x_ref.shape[1]
    lane320 = jax.lax.broadcasted_iota(jnp.int32, (1, HW + 4 * W), 1) % W

    xtall = jnp.tanh(x_ref[...])
    xt_s[...] = xtall
    xts_s[...] = (xtall * jax.nn.sigmoid(xtall)).astype(bf16)
    w_ref[...] = wf_ref[...].astype(bf16)

    C = x_ref.shape[0] // B
    cb0 = jnp.full((C, HW), 0.7310586, bf16)

    def sample(b, carry):
        xt = xt_s[pl.ds(b * C, C)]
        xt2 = xt * xt
        xts = xts_s[pl.ds(b * C, C)]

        def poly23(e_):
            b2 = 2.25 * jnp.sum(jnp.where((ri == e_) & (ci_ == 1), bv, 0.0))
            b3 = (300.0 / 9.0) * jnp.sum(
                jnp.where((ri == e_) & (ci_ == 2), bv, 0.0))
            p2 = xt2 - b2
            p3 = xt * p2 - b3 * xt
            return p2, p3

        e0 = jnp.sum(jnp.where(lane16 == 2 * b, iv, 0))
        e1 = jnp.sum(jnp.where(lane16 == 2 * b + 1, iv, 0))
        gate0 = jnp.sum(jnp.where(lane16 == 2 * b, gvv, 0.0))
        gate1 = jnp.sum(jnp.where(lane16 == 2 * b + 1, gvv, 0.0))
        p2a, p3a = poly23(e0)
        p2b, p3b = poly23(e1)
        pp = jnp.concatenate([p2a, p3a, p2b, p3b], axis=0)
        pp = (pp * jax.nn.sigmoid(pp)).astype(bf16)

        acc = jnp.zeros((o_ref.shape[1], HW), f32)
        for k in range(_K):
            e = e0 if k == 0 else e1
            gate = gate0 if k == 0 else gate1
            s23 = jax.lax.slice(pp, (2 * C * k, 0), (2 * C * (k + 1), HW))
            g = jnp.concatenate([cb0, xts, s23], axis=0)

            CI = g.shape[0]
            padz = jnp.zeros((CI, 2 * W), dtype=bf16)
            gext = jnp.concatenate([padz, g, padz], axis=1)
            gl = jnp.where(lane320 != W - 1, gext, bf16(0))
            gr = jnp.where(lane320 != 0, gext, bf16(0))

            acck = jnp.zeros((o_ref.shape[1], HW), f32)
            acck2 = jnp.zeros((o_ref.shape[1], HW), f32)
            for j in range(9):
                dy, dx = j // 3, j % 3
                off = W * (dy - 1) + (dx - 1)
                src = (gl, gext, gr)[dx]
                s = jax.lax.slice(src, (0, 2 * W + off),
                                  (CI, 2 * W + off + HW))
                d = jax.lax.dot(w_ref[e, j], s, preferred_element_type=f32)
                if j % 2 == 0:
                    acck = acck + d
                else:
                    acck2 = acck2 + d
            acc = acc + gate * (acck + acck2)

        o_ref[b] = acc
        return carry

    jax.lax.fori_loop(0, B, sample, 0)


def kernel(x, w_gate, poly_weights, beta_weights):
    B, C, H, W = x.shape
    E, O, CI, KH, KW = poly_weights.shape
    HW = H * W
    x2 = x.reshape(B * C, HW)
    pwt = jnp.transpose(poly_weights, (0, 3, 4, 1, 2)).reshape(
        E, KH * KW, O, CI)

    sc_gate = _make_sc_gate(B, C, HW, E)
    idx16, gv16, loss16 = sc_gate(x2, w_gate.reshape(-1))

    y = pl.pallas_call(
        _conv_body,
        out_shape=jax.ShapeDtypeStruct((B, O, HW), jnp.float32),
        scratch_shapes=[
            pltpu.VMEM((B * C, HW), jnp.float32),
            pltpu.VMEM((B * C, HW), jnp.bfloat16),
            pltpu.VMEM((E, KH * KW, O, CI), jnp.bfloat16),
        ],
    )(x2, idx16.reshape(1, 16), gv16.reshape(1, 16), pwt, beta_weights)

    return y.reshape(B, O, H, W), loss16[0]
